# baseline (speedup 1.0000x reference)
import functools

import jax
import jax.numpy as jnp
from jax import lax
from jax.experimental import pallas as pl
from jax.experimental.pallas import tpu as pltpu
from jax.experimental.pallas import tpu_sc as plsc

DIM = 256
K = 8192
N = 9216

BN = 512
BK = 2048
NB = N // BN
KB = K // BK


def _argmin_body(xf_ref, eT_ref, xsq_ref, esq_ref, out_ref, bv_ref, bi_ref):
    kb = pl.program_id(1)
    dot = jnp.dot(xf_ref[...], eT_ref[...], preferred_element_type=jnp.float32)
    dist = -(xsq_ref[...] - 2.0 * dot + esq_ref[...])
    m = jnp.max(dist, axis=1, keepdims=True)
    ids = lax.broadcasted_iota(jnp.int32, (BN, BK), 1) + kb * BK
    cand = jnp.where(dist == m, ids, jnp.int32(2**30))
    idx = jnp.min(cand, axis=1, keepdims=True)

    @pl.when(kb == 0)
    def _():
        bv_ref[...] = m
        bi_ref[...] = idx

    @pl.when(kb > 0)
    def _():
        better = m > bv_ref[...]
        bv_ref[...] = jnp.where(better, m, bv_ref[...])
        bi_ref[...] = jnp.where(better, idx, bi_ref[...])

    @pl.when(kb == KB - 1)
    def _():
        out_ref[0, 0, :] = bi_ref[...][:, 0]


def _nearest_indices(xf, eT, xsq, esq):
    out = pl.pallas_call(
        _argmin_body,
        grid=(NB, KB),
        in_specs=[
            pl.BlockSpec((BN, DIM), lambda i, k: (i, 0)),
            pl.BlockSpec((DIM, BK), lambda i, k: (0, k)),
            pl.BlockSpec((BN, 1), lambda i, k: (i, 0)),
            pl.BlockSpec((1, BK), lambda i, k: (0, k)),
        ],
        out_specs=pl.BlockSpec((1, 1, BN), lambda i, k: (i, 0, 0)),
        out_shape=jax.ShapeDtypeStruct((NB, 1, BN), jnp.int32),
        scratch_shapes=[
            pltpu.VMEM((BN, 1), jnp.float32),
            pltpu.VMEM((BN, 1), jnp.int32),
        ],
    )(xf, eT, xsq, esq)
    return out.reshape(N)


def _gather_rows(table, ind):
    info = plsc.get_sparse_core_info()
    nc, ns = info.num_cores, info.num_subcores
    nw = nc * ns
    b_per_w = N // nw
    mesh = plsc.VectorSubcoreMesh(core_axis_name="c", subcore_axis_name="s")

    @functools.partial(
        pl.kernel,
        mesh=mesh,
        out_type=jax.ShapeDtypeStruct((N, DIM), jnp.float32),
        scratch_types=[
            pltpu.VMEM((b_per_w,), jnp.int32),
            pltpu.VMEM((b_per_w, DIM), jnp.float32),
            pltpu.SemaphoreType.DMA,
        ],
    )
    def gather_k(table_hbm, idx_hbm, out_hbm, idx_v, rows_v, sem):
        wid = lax.axis_index("s") * nc + lax.axis_index("c")
        base = wid * b_per_w
        pltpu.sync_copy(idx_hbm.at[pl.ds(base, b_per_w)], idx_v)
        pltpu.async_copy(table_hbm.at[idx_v], rows_v, sem).wait()
        pltpu.sync_copy(rows_v, out_hbm.at[pl.ds(base, b_per_w)])

    return gather_k(table, ind)


def kernel(x, embed):
    xf = x[0]
    e0 = embed[0]
    eT = e0.T
    xsq = jnp.sum(xf * xf, axis=-1, keepdims=True)
    esq = jnp.sum(e0 * e0, axis=-1)[None, :]
    ind = _nearest_indices(xf, eT, xsq, esq)
    quantize = _gather_rows(e0, ind)
    return quantize, ind.reshape(1, N)


# speedup vs baseline: 1.3059x; 1.3059x over previous
import functools

import jax
import jax.numpy as jnp
from jax import lax
from jax.experimental import pallas as pl
from jax.experimental.pallas import tpu as pltpu
from jax.experimental.pallas import tpu_sc as plsc

DIM = 256
K = 8192
N = 9216

BN = 256
NB = N // BN


def _argmin_body(xf_ref, eT_ref, xsq_ref, esq_ref, out_ref):
    dot = jnp.dot(xf_ref[...], eT_ref[...], preferred_element_type=jnp.float32)
    t = xsq_ref[...] - 2.0 * dot + esq_ref[...]
    out_ref[0, 0, :] = jnp.argmin(t, axis=1).astype(jnp.int32)


def _nearest_indices(xf, eT, xsq, esq):
    out = pl.pallas_call(
        _argmin_body,
        grid=(NB,),
        in_specs=[
            pl.BlockSpec((BN, DIM), lambda i: (i, 0)),
            pl.BlockSpec((DIM, K), lambda i: (0, 0)),
            pl.BlockSpec((BN, 1), lambda i: (i, 0)),
            pl.BlockSpec((1, K), lambda i: (0, 0)),
        ],
        out_specs=pl.BlockSpec((1, 1, BN), lambda i: (i, 0, 0)),
        out_shape=jax.ShapeDtypeStruct((NB, 1, BN), jnp.int32),
    )(xf, eT, xsq, esq)
    return out.reshape(N)


def _gather_rows(table, ind):
    info = plsc.get_sparse_core_info()
    nc, ns = info.num_cores, info.num_subcores
    nw = nc * ns
    b_per_w = N // nw
    mesh = plsc.VectorSubcoreMesh(core_axis_name="c", subcore_axis_name="s")

    @functools.partial(
        pl.kernel,
        mesh=mesh,
        out_type=jax.ShapeDtypeStruct((N, DIM), jnp.float32),
        scratch_types=[
            pltpu.VMEM((b_per_w,), jnp.int32),
            pltpu.VMEM((b_per_w, DIM), jnp.float32),
            pltpu.SemaphoreType.DMA,
        ],
    )
    def gather_k(table_hbm, idx_hbm, out_hbm, idx_v, rows_v, sem):
        wid = lax.axis_index("s") * nc + lax.axis_index("c")
        base = wid * b_per_w
        pltpu.sync_copy(idx_hbm.at[pl.ds(base, b_per_w)], idx_v)
        pltpu.async_copy(table_hbm.at[idx_v], rows_v, sem).wait()
        pltpu.sync_copy(rows_v, out_hbm.at[pl.ds(base, b_per_w)])

    return gather_k(table, ind)


def kernel(x, embed):
    xf = x[0]
    e0 = embed[0]
    eT = e0.T
    xsq = jnp.sum(xf * xf, axis=-1, keepdims=True)
    esq = jnp.sum(e0 * e0, axis=-1)[None, :]
    ind = _nearest_indices(xf, eT, xsq, esq)
    quantize = _gather_rows(e0, ind)
    return quantize, ind.reshape(1, N)


# speedup vs baseline: 1.3712x; 1.0500x over previous
import functools

import jax
import jax.numpy as jnp
from jax import lax
from jax.experimental import pallas as pl
from jax.experimental.pallas import tpu as pltpu
from jax.experimental.pallas import tpu_sc as plsc

DIM = 256
K = 8192
N = 9216

BN = 256
NB = N // BN


def _argmin_body(xf_ref, e_ref, xsq_ref, esq_ref, out_ref):
    dot = lax.dot_general(
        xf_ref[...], e_ref[...],
        dimension_numbers=(((1,), (1,)), ((), ())),
        preferred_element_type=jnp.float32,
    )
    t = xsq_ref[...] - 2.0 * dot + esq_ref[...]
    out_ref[0, 0, :] = jnp.argmin(t, axis=1).astype(jnp.int32)


def _nearest_indices(xf, eT, xsq, esq):
    out = pl.pallas_call(
        _argmin_body,
        grid=(NB,),
        in_specs=[
            pl.BlockSpec((BN, DIM), lambda i: (i, 0)),
            pl.BlockSpec((K, DIM), lambda i: (0, 0)),
            pl.BlockSpec((BN, 1), lambda i: (i, 0)),
            pl.BlockSpec((1, K), lambda i: (0, 0)),
        ],
        out_specs=pl.BlockSpec((1, 1, BN), lambda i: (i, 0, 0)),
        out_shape=jax.ShapeDtypeStruct((NB, 1, BN), jnp.int32),
    )(xf, eT, xsq, esq)
    return out.reshape(N)


def _gather_rows(table, ind):
    info = plsc.get_sparse_core_info()
    nc, ns = info.num_cores, info.num_subcores
    nw = nc * ns
    b_per_w = N // nw
    mesh = plsc.VectorSubcoreMesh(core_axis_name="c", subcore_axis_name="s")

    @functools.partial(
        pl.kernel,
        mesh=mesh,
        out_type=jax.ShapeDtypeStruct((N, DIM), jnp.float32),
        scratch_types=[
            pltpu.VMEM((b_per_w,), jnp.int32),
            pltpu.VMEM((b_per_w, DIM), jnp.float32),
            pltpu.SemaphoreType.DMA,
        ],
    )
    def gather_k(table_hbm, idx_hbm, out_hbm, idx_v, rows_v, sem):
        wid = lax.axis_index("s") * nc + lax.axis_index("c")
        base = wid * b_per_w
        pltpu.sync_copy(idx_hbm.at[pl.ds(base, b_per_w)], idx_v)
        pltpu.async_copy(table_hbm.at[idx_v], rows_v, sem).wait()
        pltpu.sync_copy(rows_v, out_hbm.at[pl.ds(base, b_per_w)])

    return gather_k(table, ind)


def kernel(x, embed):
    xf = x[0]
    e0 = embed[0]
    xsq = jnp.sum(xf * xf, axis=-1, keepdims=True)
    esq = jnp.sum(e0 * e0, axis=-1)[None, :]
    ind = _nearest_indices(xf, e0, xsq, esq)
    quantize = _gather_rows(e0, ind)
    return quantize, ind.reshape(1, N)


# speedup vs baseline: 1.5659x; 1.1420x over previous
import functools

import jax
import jax.numpy as jnp
from jax import lax
from jax.experimental import pallas as pl
from jax.experimental.pallas import tpu as pltpu
from jax.experimental.pallas import tpu_sc as plsc

DIM = 256
K = 8192
N = 9216

BN = 256
NB = N // BN


def _argmin_body(xf_ref, e_ref, xsq_ref, esq_ref, out_ref):
    dot2 = lax.dot_general(
        xf_ref[...] * 2.0, e_ref[...],
        dimension_numbers=(((1,), (1,)), ((), ())),
        preferred_element_type=jnp.float32,
    )
    t = xsq_ref[...] - dot2 + esq_ref[...]
    out_ref[0, 0, :] = jnp.argmin(t, axis=1).astype(jnp.int32)


def _nearest_indices(xf, eT, xsq, esq):
    out = pl.pallas_call(
        _argmin_body,
        grid=(NB,),
        in_specs=[
            pl.BlockSpec((BN, DIM), lambda i: (i, 0)),
            pl.BlockSpec((K, DIM), lambda i: (0, 0)),
            pl.BlockSpec((BN, 1), lambda i: (i, 0)),
            pl.BlockSpec((1, K), lambda i: (0, 0)),
        ],
        out_specs=pl.BlockSpec((1, 1, BN), lambda i: (i, 0, 0)),
        out_shape=jax.ShapeDtypeStruct((NB, 1, BN), jnp.int32),
    )(xf, eT, xsq, esq)
    return out.reshape(N)


def _gather_rows(table, ind):
    info = plsc.get_sparse_core_info()
    nc, ns = info.num_cores, info.num_subcores
    nw = nc * ns
    b_per_w = N // nw
    mesh = plsc.VectorSubcoreMesh(core_axis_name="c", subcore_axis_name="s")

    @functools.partial(
        pl.kernel,
        mesh=mesh,
        out_type=jax.ShapeDtypeStruct((N, DIM), jnp.float32),
        scratch_types=[
            pltpu.VMEM((b_per_w,), jnp.int32),
            pltpu.VMEM((b_per_w, DIM), jnp.float32),
            pltpu.SemaphoreType.DMA,
        ],
    )
    def gather_k(table_hbm, idx_hbm, out_hbm, idx_v, rows_v, sem):
        wid = lax.axis_index("s") * nc + lax.axis_index("c")
        base = wid * b_per_w
        pltpu.sync_copy(idx_hbm.at[pl.ds(base, b_per_w)], idx_v)
        pltpu.async_copy(table_hbm.at[idx_v], rows_v, sem).wait()
        pltpu.sync_copy(rows_v, out_hbm.at[pl.ds(base, b_per_w)])

    return gather_k(table, ind)


def kernel(x, embed):
    xf = x[0]
    e0 = embed[0]
    xsq = jnp.sum(xf * xf, axis=-1, keepdims=True)
    esq = jnp.sum(e0 * e0, axis=-1)[None, :]
    ind = _nearest_indices(xf, e0, xsq, esq)
    quantize = _gather_rows(e0, ind)
    return quantize, ind.reshape(1, N)


# speedup vs baseline: 1.7232x; 1.1004x over previous
import functools

import jax
import jax.numpy as jnp
from jax import lax
from jax.experimental import pallas as pl
from jax.experimental.pallas import tpu as pltpu
from jax.experimental.pallas import tpu_sc as plsc

DIM = 256
K = 8192
N = 9216

BN = 512
NB = N // BN


def _argmin_body(xf_ref, e_ref, xsq_ref, esq_ref, out_ref):
    dot2 = lax.dot_general(
        xf_ref[...] * 2.0, e_ref[...],
        dimension_numbers=(((1,), (1,)), ((), ())),
        preferred_element_type=jnp.float32,
    )
    t = xsq_ref[...] - dot2 + esq_ref[...]
    out_ref[0, 0, :] = jnp.argmin(t, axis=1).astype(jnp.int32)


def _nearest_indices(xf, eT, xsq, esq):
    out = pl.pallas_call(
        _argmin_body,
        grid=(NB,),
        in_specs=[
            pl.BlockSpec((BN, DIM), lambda i: (i, 0)),
            pl.BlockSpec((K, DIM), lambda i: (0, 0)),
            pl.BlockSpec((BN, 1), lambda i: (i, 0)),
            pl.BlockSpec((1, K), lambda i: (0, 0)),
        ],
        out_specs=pl.BlockSpec((1, 1, BN), lambda i: (i, 0, 0)),
        out_shape=jax.ShapeDtypeStruct((NB, 1, BN), jnp.int32),
    )(xf, eT, xsq, esq)
    return out.reshape(N)


def _gather_rows(table, ind):
    info = plsc.get_sparse_core_info()
    nc, ns = info.num_cores, info.num_subcores
    nw = nc * ns
    b_per_w = N // nw
    mesh = plsc.VectorSubcoreMesh(core_axis_name="c", subcore_axis_name="s")

    @functools.partial(
        pl.kernel,
        mesh=mesh,
        out_type=jax.ShapeDtypeStruct((N, DIM), jnp.float32),
        scratch_types=[
            pltpu.VMEM((b_per_w,), jnp.int32),
            pltpu.VMEM((b_per_w, DIM), jnp.float32),
            pltpu.SemaphoreType.DMA,
        ],
    )
    def gather_k(table_hbm, idx_hbm, out_hbm, idx_v, rows_v, sem):
        wid = lax.axis_index("s") * nc + lax.axis_index("c")
        base = wid * b_per_w
        pltpu.sync_copy(idx_hbm.at[pl.ds(base, b_per_w)], idx_v)
        pltpu.async_copy(table_hbm.at[idx_v], rows_v, sem).wait()
        pltpu.sync_copy(rows_v, out_hbm.at[pl.ds(base, b_per_w)])

    return gather_k(table, ind)


def kernel(x, embed):
    xf = x[0]
    e0 = embed[0]
    xsq = jnp.sum(xf * xf, axis=-1, keepdims=True)
    esq = jnp.sum(e0 * e0, axis=-1)[None, :]
    ind = _nearest_indices(xf, e0, xsq, esq)
    quantize = _gather_rows(e0, ind)
    return quantize, ind.reshape(1, N)
